# baseline (speedup 1.0000x reference)
import jax
import jax.numpy as jnp
from jax.experimental import pallas as pl
from jax.experimental.pallas import tpu as pltpu

_EPS = 1e-12


def _embed_ln_kernel(x_ref, tt_ref, pos_ref, seg_ref, gamma_ref, beta_ref,
                     out_ref):
    x = x_ref[...]
    pos = pos_ref[...]
    tt = tt_ref[...]
    seg = jnp.where(tt == 0, seg_ref[0:1, :], seg_ref[1:2, :])
    e = x + pos + seg
    d = e.shape[-1]
    s1 = jnp.sum(e, axis=-1, keepdims=True)
    s2 = jnp.sum(e * e, axis=-1, keepdims=True)
    mean = s1 * (1.0 / d)
    var = s2 * (1.0 / d) - mean * mean
    inv = jax.lax.rsqrt(var + _EPS)
    scale = inv * gamma_ref[...]
    shift = beta_ref[...] - mean * scale
    out_ref[...] = e * scale + shift


def kernel(inputs_embeds, token_type_ids, pos_table, seg_table, gamma, beta):
    B, S, D = inputs_embeds.shape
    BS = 2048
    n_s = S // BS

    x2 = inputs_embeds.reshape(B * S, D)
    tt2 = token_type_ids.astype(jnp.int32).reshape(B * S, 1)
    gamma2 = gamma.reshape(1, D)
    beta2 = beta.reshape(1, D)

    out = pl.pallas_call(
        _embed_ln_kernel,
        grid=(n_s, B),
        in_specs=[
            pl.BlockSpec((BS, D), lambda s, b: (b * n_s + s, 0)),
            pl.BlockSpec((BS, 1), lambda s, b: (b * n_s + s, 0)),
            pl.BlockSpec((BS, D), lambda s, b: (s, 0)),
            pl.BlockSpec((2, D), lambda s, b: (0, 0)),
            pl.BlockSpec((1, D), lambda s, b: (0, 0)),
            pl.BlockSpec((1, D), lambda s, b: (0, 0)),
        ],
        out_specs=pl.BlockSpec((BS, D), lambda s, b: (b * n_s + s, 0)),
        out_shape=jax.ShapeDtypeStruct((B * S, D), jnp.float32),
        compiler_params=pltpu.CompilerParams(
            dimension_semantics=("parallel", "parallel"),
            vmem_limit_bytes=120 * 1024 * 1024),
    )(x2, tt2, pos_table[:S], seg_table, gamma2, beta2)

    return out.reshape(B, S, D)


# speedup vs baseline: 1.0330x; 1.0330x over previous
import jax
import jax.numpy as jnp
from jax.experimental import pallas as pl
from jax.experimental.pallas import tpu as pltpu

_EPS = 1e-12


def _embed_ln_kernel(x_ref, tt_ref, pos_ref, seg_ref, gamma_ref, beta_ref,
                     out_ref):
    s = pl.program_id(0)
    x = x_ref[...]
    pos = pos_ref[pl.ds(s * x.shape[0], x.shape[0]), :]
    tt = tt_ref[...]
    seg = jnp.where(tt == 0, seg_ref[0:1, :], seg_ref[1:2, :])
    e = x + pos + seg
    d = e.shape[-1]
    s1 = jnp.sum(e, axis=-1, keepdims=True)
    s2 = jnp.sum(e * e, axis=-1, keepdims=True)
    mean = s1 * (1.0 / d)
    var = s2 * (1.0 / d) - mean * mean
    inv = jax.lax.rsqrt(var + _EPS)
    scale = inv * gamma_ref[...]
    shift = beta_ref[...] - mean * scale
    out_ref[...] = e * scale + shift


def kernel(inputs_embeds, token_type_ids, pos_table, seg_table, gamma, beta):
    B, S, D = inputs_embeds.shape
    BS = 2048
    n_s = S // BS

    x2 = inputs_embeds.reshape(B * S, D)
    tt2 = token_type_ids.astype(jnp.int32).reshape(B * S, 1)
    gamma2 = gamma.reshape(1, D)
    beta2 = beta.reshape(1, D)

    out = pl.pallas_call(
        _embed_ln_kernel,
        grid=(n_s, B),
        in_specs=[
            pl.BlockSpec((BS, D), lambda s, b: (b * n_s + s, 0)),
            pl.BlockSpec((BS, 1), lambda s, b: (b * n_s + s, 0)),
            pl.BlockSpec((S, D), lambda s, b: (0, 0)),
            pl.BlockSpec((2, D), lambda s, b: (0, 0)),
            pl.BlockSpec((1, D), lambda s, b: (0, 0)),
            pl.BlockSpec((1, D), lambda s, b: (0, 0)),
        ],
        out_specs=pl.BlockSpec((BS, D), lambda s, b: (b * n_s + s, 0)),
        out_shape=jax.ShapeDtypeStruct((B * S, D), jnp.float32),
        compiler_params=pltpu.CompilerParams(
            dimension_semantics=("parallel", "parallel"),
            vmem_limit_bytes=120 * 1024 * 1024),
    )(x2, tt2, pos_table[:S], seg_table, gamma2, beta2)

    return out.reshape(B, S, D)
